# Initial kernel scaffold; baseline (speedup 1.0000x reference)
import jax
import jax.numpy as jnp
from jax.experimental import pallas as pl


def kernel(x, edge_index, W1l, W1r, att1, b1, W2l, W2r, att2, b2):
    raise NotImplementedError("write your pallas kernel here")


# speedup vs baseline: 12.4371x; 12.4371x over previous
import functools

import jax
import jax.numpy as jnp
from jax import lax
from jax.experimental import pallas as pl
from jax.experimental.pallas import tpu as pltpu
from jax.experimental.pallas import tpu_sc as plsc

N = 10000
E = 320000
D_IN = 128
D_H = 16
HEADS = 8
D_OUT = 47

NPAD = 10112
EPAD = 331776
CHUNK = 128
K1 = EPAD // (16 * CHUNK)
K2 = EPAD // (32 * CHUNK)
STRIPE = NPAD // 16


def _leaky(u):
    return jnp.maximum(u, 0.0) + 0.2 * jnp.minimum(u, 0.0)


def _stage_a_body(x_ref, wl_ref, wr_ref, att_ref, xl_ref, rm_ref):
    x = x_ref[...]
    xl = jnp.dot(x, wl_ref[...], preferred_element_type=jnp.float32)
    xr = jnp.dot(x, wr_ref[...], preferred_element_type=jnp.float32)
    t = _leaky(xl + xr)
    att = att_ref[...]
    mts = [
        jnp.sum(t[:, h * 16:(h + 1) * 16] * att[h][None, :], axis=-1,
                keepdims=True)
        for h in range(HEADS)
    ]
    xl_ref[0] = xl[:, :64]
    xl_ref[1] = xl[:, 64:]
    zero4 = jnp.zeros((NPAD, 4), jnp.float32)
    for c in range(2):
        rm_ref[c] = jnp.concatenate(
            [xr[:, c * 64:(c + 1) * 64]] + mts[c * 4:(c + 1) * 4] + [zero4],
            axis=1)


def _stage_a(xp, w1l, w1r, att1):
    xl, rm = pl.pallas_call(
        _stage_a_body,
        compiler_params=pltpu.CompilerParams(
            vmem_limit_bytes=110 * 1024 * 1024),
        out_shape=(
            jax.ShapeDtypeStruct((2, NPAD, 64), jnp.float32),
            jax.ShapeDtypeStruct((2, NPAD, 72), jnp.float32),
        ),
    )(xp, w1l, w1r, att1)
    return xl.reshape(2 * NPAD, 64), rm.reshape(2 * NPAD, 72)


def _stage_b_body(acc_ref, b1_ref, w2l_ref, w2r_ref, att2_ref,
                  xl2_ref, rm2_ref):
    hs = []
    for c in range(2):
        acc = acc_ref[c]
        for h in range(4):
            z = acc[:, 64 + h:65 + h] + 1e-16
            hs.append(acc[:, h * 16:(h + 1) * 16] / z)
    h1 = jnp.concatenate(hs, axis=1) + b1_ref[...][None, :]
    h1 = jnp.where(h1 > 0, h1, jnp.exp(jnp.minimum(h1, 0.0)) - 1.0)
    xl2 = jnp.dot(h1, w2l_ref[...], preferred_element_type=jnp.float32)
    xr2 = jnp.dot(h1, w2r_ref[...], preferred_element_type=jnp.float32)
    mt2 = jnp.sum(_leaky(xl2 + xr2) * att2_ref[...], axis=-1, keepdims=True)
    xl2_ref[...] = jnp.concatenate([xl2, jnp.ones((NPAD, 1), jnp.float32)],
                                   axis=1)
    rm2_ref[...] = jnp.concatenate([xr2, mt2], axis=1)


def _stage_b(acc1, b1, w2l, w2r, att2):
    return pl.pallas_call(
        _stage_b_body,
        compiler_params=pltpu.CompilerParams(
            vmem_limit_bytes=110 * 1024 * 1024),
        out_shape=(
            jax.ShapeDtypeStruct((NPAD, 48), jnp.float32),
            jax.ShapeDtypeStruct((NPAD, 48), jnp.float32),
        ),
    )(acc1.reshape(2, NPAD, 72), b1, w2l, w2r, att2)


def _stage_c_body(acc_ref, b2_ref, out_ref):
    s = acc_ref[0] + acc_ref[1]
    o = s[:, :47] / (s[:, 47:48] + 1e-16) + b2_ref[...][None, :]
    m = jnp.max(o, axis=1, keepdims=True)
    lse = jnp.log(jnp.sum(jnp.exp(o - m), axis=1, keepdims=True)) + m
    out_ref[...] = o - lse


def _stage_c(acc2, b2):
    return pl.pallas_call(
        _stage_c_body,
        compiler_params=pltpu.CompilerParams(
            vmem_limit_bytes=110 * 1024 * 1024),
        out_shape=jax.ShapeDtypeStruct((NPAD, 47), jnp.float32),
    )(acc2.reshape(2, NPAD, 48), b2)


def _make_edge_kernel(dl, dr, heads, zcol, n_chunks, split_edges_by_core,
                      core_tbl_rows, att_rows_per_core):
    mesh = plsc.VectorSubcoreMesh(core_axis_name="c", subcore_axis_name="s")
    ngroups = CHUNK // 16
    mul_cols = zcol if zcol + heads <= dl else dl
    cph = dl // heads

    @functools.partial(
        pl.kernel,
        mesh=mesh,
        compiler_params=pltpu.CompilerParams(needs_layout_passes=False,
                                             use_tc_tiling_on_sc=False),
        out_type=jax.ShapeDtypeStruct((2 * NPAD, dr), jnp.float32),
        scratch_types=[
            pltpu.VMEM_SHARED((NPAD, dr), jnp.float32),
            pltpu.VMEM((CHUNK,), jnp.int32),
            pltpu.VMEM((CHUNK,), jnp.int32),
            pltpu.VMEM((CHUNK,), jnp.int32),
            pltpu.VMEM((CHUNK, dl), jnp.float32),
            pltpu.VMEM((CHUNK, dr), jnp.float32),
            pltpu.VMEM((CHUNK, dr), jnp.float32),
            pltpu.VMEM((dl, 16), jnp.float32),
        ],
    )
    def edge_kernel(xl_hbm, rm_hbm, src_hbm, dst_hbm, attb_hbm, zeros_hbm,
                    acc_out, acc_sh, srcb, dstb, dstb2, rows_l, rows_r, outb,
                    attv):
        c = lax.axis_index("c")
        s = lax.axis_index("s")
        tbl_off = c * core_tbl_rows
        pltpu.sync_copy(attb_hbm.at[pl.ds(c * att_rows_per_core, dl)], attv)
        pltpu.sync_copy(zeros_hbm, acc_sh.at[pl.ds(s * STRIPE, STRIPE)])
        plsc.subcore_barrier()

        if split_edges_by_core:
            ebase0 = (s * 2 + c) * (n_chunks * CHUNK)
        else:
            ebase0 = s * (n_chunks * CHUNK)
        iot = lax.iota(jnp.int32, 16)

        def chunk_body(i, _):
            base = ebase0 + i * CHUNK
            pltpu.sync_copy(src_hbm.at[pl.ds(base, CHUNK)], srcb)
            pltpu.sync_copy(dst_hbm.at[pl.ds(base, CHUNK)], dstb)
            for j in range(ngroups):
                sl = pl.ds(j * 16, 16)
                srcb[sl] = srcb[sl] + tbl_off
                dstb2[sl] = dstb[sl] + tbl_off
            pltpu.sync_copy(xl_hbm.at[srcb], rows_l)
            pltpu.sync_copy(rm_hbm.at[dstb2], rows_r)
            for g in range(ngroups):
                rowi = iot + g * 16
                accs = [jnp.zeros((16,), jnp.float32) for _ in range(heads)]
                for d in range(dl):
                    cd = jnp.full((16,), d, jnp.int32)
                    vl = plsc.load_gather(rows_l, [rowi, cd])
                    vr = plsc.load_gather(rows_r, [rowi, cd])
                    u = vl + vr
                    t = jnp.maximum(u, 0.0) + 0.2 * jnp.minimum(u, 0.0)
                    accs[d // cph] = accs[d // cph] + t * attv[d]
                avs = []
                for h in range(heads):
                    mt = plsc.load_gather(
                        rows_r, [rowi, jnp.full((16,), zcol + h, jnp.int32)])
                    avs.append(jnp.exp(accs[h] - mt))
                for d in range(mul_cols):
                    cd = jnp.full((16,), d, jnp.int32)
                    vl = plsc.load_gather(rows_l, [rowi, cd])
                    plsc.store_scatter(outb, [rowi, cd], avs[d // cph] * vl)
                if mul_cols < zcol + heads:
                    for h in range(heads):
                        ch = jnp.full((16,), zcol + h, jnp.int32)
                        plsc.store_scatter(outb, [rowi, ch], avs[h])
            pltpu.sync_copy(outb, acc_sh.at[dstb], add=True)
            return 0

        lax.fori_loop(0, n_chunks, chunk_body, 0)
        plsc.subcore_barrier()
        pltpu.sync_copy(acc_sh.at[pl.ds(s * STRIPE, STRIPE)],
                        acc_out.at[pl.ds(c * NPAD + s * STRIPE, STRIPE)])

    return edge_kernel


_edge_l1 = _make_edge_kernel(dl=64, dr=72, heads=4, zcol=64, n_chunks=K1,
                             split_edges_by_core=False, core_tbl_rows=NPAD,
                             att_rows_per_core=64)
_edge_l2 = _make_edge_kernel(dl=48, dr=48, heads=1, zcol=47, n_chunks=K2,
                             split_edges_by_core=True, core_tbl_rows=0,
                             att_rows_per_core=0)


def kernel(x, edge_index, W1l, W1r, att1, b1, W2l, W2r, att2, b2):
    loop = jnp.arange(N, dtype=edge_index.dtype)
    src = jnp.pad(jnp.concatenate([edge_index[0], loop]), (0, EPAD - E - N),
                  constant_values=N)
    dst = jnp.pad(jnp.concatenate([edge_index[1], loop]), (0, EPAD - E - N),
                  constant_values=N)
    xp = jnp.pad(x, ((0, NPAD - N), (0, 0)))

    attb1 = jnp.broadcast_to(att1.reshape(128, 1), (128, 16))
    att2p = jnp.concatenate([att2[0], jnp.zeros((1,), jnp.float32)])
    attb2 = jnp.broadcast_to(att2p.reshape(48, 1), (48, 16))

    zeros72 = jnp.zeros((STRIPE, 72), jnp.float32)
    zeros48 = jnp.zeros((STRIPE, 48), jnp.float32)

    xl_tbl, rm_tbl = _stage_a(xp, W1l, W1r, att1)
    acc1 = _edge_l1(xl_tbl, rm_tbl, src, dst, attb1, zeros72)
    xl2_tbl, rm2_tbl = _stage_b(acc1, b1, W2l, W2r, att2)
    acc2 = _edge_l2(xl2_tbl, rm2_tbl, src, dst, attb2, zeros48)
    out = _stage_c(acc2, b2)
    return out[:N]


# speedup vs baseline: 18.6590x; 1.5003x over previous
import functools

import jax
import jax.numpy as jnp
from jax import lax
from jax.experimental import pallas as pl
from jax.experimental.pallas import tpu as pltpu
from jax.experimental.pallas import tpu_sc as plsc

N = 10000
E = 320000
D_IN = 128
D_H = 16
HEADS = 8
D_OUT = 47

NPAD = 10112
EPAD = 335872
CHUNK = 128
K1 = EPAD // (16 * CHUNK)
K2 = EPAD // (32 * CHUNK)
STRIPE = NPAD // 16


def _leaky(u):
    return jnp.maximum(u, 0.0) + 0.2 * jnp.minimum(u, 0.0)


def _stage_a_body(x_ref, wl_ref, wr_ref, att_ref, xl_ref, rm_ref):
    x = x_ref[...]
    xl = jnp.dot(x, wl_ref[...], preferred_element_type=jnp.float32)
    xr = jnp.dot(x, wr_ref[...], preferred_element_type=jnp.float32)
    t = _leaky(xl + xr)
    att = att_ref[...]
    mts = [
        jnp.sum(t[:, h * 16:(h + 1) * 16] * att[h][None, :], axis=-1,
                keepdims=True)
        for h in range(HEADS)
    ]
    xl_ref[0] = xl[:, :64]
    xl_ref[1] = xl[:, 64:]
    zero4 = jnp.zeros((NPAD, 4), jnp.float32)
    for c in range(2):
        rm_ref[c] = jnp.concatenate(
            [xr[:, c * 64:(c + 1) * 64]] + mts[c * 4:(c + 1) * 4] + [zero4],
            axis=1)


def _stage_a(xp, w1l, w1r, att1):
    xl, rm = pl.pallas_call(
        _stage_a_body,
        compiler_params=pltpu.CompilerParams(
            vmem_limit_bytes=110 * 1024 * 1024),
        out_shape=(
            jax.ShapeDtypeStruct((2, NPAD, 64), jnp.float32),
            jax.ShapeDtypeStruct((2, NPAD, 72), jnp.float32),
        ),
    )(xp, w1l, w1r, att1)
    return xl.reshape(2 * NPAD, 64), rm.reshape(2 * NPAD, 72)


def _stage_b_body(acc_ref, b1_ref, w2l_ref, w2r_ref, att2_ref,
                  xl2_ref, rm2_ref):
    hs = []
    for c in range(2):
        acc = acc_ref[c]
        for h in range(4):
            z = acc[:, 64 + h:65 + h] + 1e-16
            hs.append(acc[:, h * 16:(h + 1) * 16] / z)
    h1 = jnp.concatenate(hs, axis=1) + b1_ref[...][None, :]
    h1 = jnp.where(h1 > 0, h1, jnp.exp(jnp.minimum(h1, 0.0)) - 1.0)
    xl2 = jnp.dot(h1, w2l_ref[...], preferred_element_type=jnp.float32)
    xr2 = jnp.dot(h1, w2r_ref[...], preferred_element_type=jnp.float32)
    mt2 = jnp.sum(_leaky(xl2 + xr2) * att2_ref[...], axis=-1, keepdims=True)
    xl2_ref[...] = jnp.concatenate([xl2, jnp.ones((NPAD, 1), jnp.float32)],
                                   axis=1)
    rm2_ref[...] = jnp.concatenate([xr2, mt2], axis=1)


def _stage_b(acc1, b1, w2l, w2r, att2):
    return pl.pallas_call(
        _stage_b_body,
        compiler_params=pltpu.CompilerParams(
            vmem_limit_bytes=110 * 1024 * 1024),
        out_shape=(
            jax.ShapeDtypeStruct((NPAD, 48), jnp.float32),
            jax.ShapeDtypeStruct((NPAD, 48), jnp.float32),
        ),
    )(acc1.reshape(2, NPAD, 72), b1, w2l, w2r, att2)


def _stage_c_body(acc_ref, b2_ref, out_ref):
    s = acc_ref[0] + acc_ref[1]
    o = s[:, :47] / (s[:, 47:48] + 1e-16) + b2_ref[...][None, :]
    m = jnp.max(o, axis=1, keepdims=True)
    lse = jnp.log(jnp.sum(jnp.exp(o - m), axis=1, keepdims=True)) + m
    out_ref[...] = o - lse


def _stage_c(acc2, b2):
    return pl.pallas_call(
        _stage_c_body,
        compiler_params=pltpu.CompilerParams(
            vmem_limit_bytes=110 * 1024 * 1024),
        out_shape=jax.ShapeDtypeStruct((NPAD, 47), jnp.float32),
    )(acc2.reshape(2, NPAD, 48), b2)


def _make_edge_kernel(dl, dr, heads, zcol, n_chunks, split_edges_by_core,
                      idx_rows_per_core, att_rows_per_core):
    mesh = plsc.VectorSubcoreMesh(core_axis_name="c", subcore_axis_name="s")
    ngroups = CHUNK // 16
    mul_cols = zcol if zcol + heads <= dl else dl
    cph = dl // heads

    @functools.partial(
        pl.kernel,
        mesh=mesh,
        compiler_params=pltpu.CompilerParams(needs_layout_passes=False,
                                             use_tc_tiling_on_sc=False),
        out_type=jax.ShapeDtypeStruct((2 * NPAD, dr), jnp.float32),
        scratch_types=[
            pltpu.VMEM_SHARED((NPAD, dr), jnp.float32),
            pltpu.VMEM((2, 3, CHUNK), jnp.int32),
            pltpu.VMEM((2, 1, CHUNK), jnp.int32),
            pltpu.VMEM((2, CHUNK, dl), jnp.float32),
            pltpu.VMEM((2, CHUNK, dr), jnp.float32),
            pltpu.VMEM((2, CHUNK, dr), jnp.float32),
            pltpu.VMEM((dl, 16), jnp.float32),
            pltpu.SemaphoreType.DMA,
            pltpu.SemaphoreType.DMA,
        ],
    )
    def edge_kernel(xl_hbm, rm_hbm, eidx_hbm, attb_hbm, zeros_hbm,
                    acc_out, acc_sh, idxb, sidx, rows_l, rows_r, outb,
                    attv, sem_g, sem_s):
        c = lax.axis_index("c")
        s = lax.axis_index("s")
        pltpu.sync_copy(attb_hbm.at[pl.ds(c * att_rows_per_core, dl)], attv)
        pltpu.sync_copy(zeros_hbm, acc_sh.at[pl.ds(s * STRIPE, STRIPE)])
        plsc.subcore_barrier()

        irow0 = c * idx_rows_per_core
        if split_edges_by_core:
            ebase0 = (s * 2 + c) * (n_chunks * CHUNK)
        else:
            ebase0 = s * (n_chunks * CHUNK)
        iot = lax.iota(jnp.int32, 16)
        ones = jnp.full((16,), 1, jnp.int32)

        def load_idx(i, par):
            pltpu.sync_copy(
                eidx_hbm.at[pl.ds(irow0, 3),
                            pl.ds(ebase0 + i * CHUNK, CHUNK)],
                idxb.at[par])

        def start_gathers(par):
            ib = idxb.at[par]
            pltpu.async_copy(xl_hbm.at[ib.at[0]], rows_l.at[par], sem_g)
            pltpu.async_copy(rm_hbm.at[ib.at[1]], rows_r.at[par], sem_g)

        def wait_gathers(par):
            pltpu.make_async_copy(xl_hbm.at[pl.ds(0, CHUNK)],
                                  rows_l.at[par], sem_g).wait()
            pltpu.make_async_copy(rm_hbm.at[pl.ds(0, CHUNK)],
                                  rows_r.at[par], sem_g).wait()

        def wait_scatter(par):
            pltpu.make_async_copy(rm_hbm.at[pl.ds(0, CHUNK)],
                                  outb.at[par], sem_s).wait()

        load_idx(0, 0)
        start_gathers(0)

        def chunk_body(i, _):
            par = lax.rem(i, 2)
            wait_gathers(par)

            @pl.when(i >= 2)
            def _():
                wait_scatter(par)

            for j in range(ngroups):
                sl = pl.ds(j * 16, 16)
                sidx[par, 0, sl] = idxb[par, 2, sl]

            @pl.when(i + 1 < n_chunks)
            def _():
                load_idx(i + 1, 1 - par)
                start_gathers(1 - par)

            rl = rows_l.at[par]
            rr = rows_r.at[par]
            ob = outb.at[par]

            def group_body(g, _):
                rowi = iot + g * 16
                accs = [jnp.zeros((16,), jnp.float32) for _ in range(heads)]
                for d in range(dl):
                    cd = jnp.full((16,), d, jnp.int32)
                    vl = plsc.load_gather(rl, [rowi, cd])
                    vr = plsc.load_gather(rr, [rowi, cd])
                    u = vl + vr
                    t = jnp.maximum(u, 0.0) + 0.2 * jnp.minimum(u, 0.0)
                    accs[d // cph] = accs[d // cph] + t * attv[d]
                avs = []
                for h in range(heads):
                    mt = plsc.load_gather(
                        rr, [rowi, jnp.full((16,), zcol + h, jnp.int32)])
                    avs.append(jnp.exp(accs[h] - mt))
                for d in range(mul_cols):
                    cd = jnp.full((16,), d, jnp.int32)
                    vl = plsc.load_gather(rl, [rowi, cd])
                    plsc.store_scatter(ob, [rowi, cd], avs[d // cph] * vl)
                if mul_cols < zcol + heads:
                    for h in range(heads):
                        ch = jnp.full((16,), zcol + h, jnp.int32)
                        plsc.store_scatter(ob, [rowi, ch], avs[h])
                return 0

            lax.fori_loop(0, ngroups, group_body, 0)
            pltpu.async_copy(ob, acc_sh.at[sidx.at[par].at[0]], sem_s,
                             add=True)
            return 0

        lax.fori_loop(0, n_chunks, chunk_body, 0)
        wait_scatter(0)
        wait_scatter(1)
        plsc.subcore_barrier()
        pltpu.sync_copy(acc_sh.at[pl.ds(s * STRIPE, STRIPE)],
                        acc_out.at[pl.ds(c * NPAD + s * STRIPE, STRIPE)])

    return edge_kernel


_edge_l1 = _make_edge_kernel(dl=64, dr=72, heads=4, zcol=64, n_chunks=K1,
                             split_edges_by_core=False, idx_rows_per_core=3,
                             att_rows_per_core=64)
_edge_l2 = _make_edge_kernel(dl=48, dr=48, heads=1, zcol=47, n_chunks=K2,
                             split_edges_by_core=True, idx_rows_per_core=0,
                             att_rows_per_core=0)


def kernel(x, edge_index, W1l, W1r, att1, b1, W2l, W2r, att2, b2):
    loop = jnp.arange(N, dtype=edge_index.dtype)
    src = jnp.pad(jnp.concatenate([edge_index[0], loop]), (0, EPAD - E - N),
                  constant_values=N)
    dst = jnp.pad(jnp.concatenate([edge_index[1], loop]), (0, EPAD - E - N),
                  constant_values=N)
    xp = jnp.pad(x, ((0, NPAD - N), (0, 0)))

    eidx1 = jnp.stack([src, dst, dst, src + NPAD, dst + NPAD, dst])
    eidx2 = jnp.stack([src, dst, dst])

    attb1 = jnp.broadcast_to(att1.reshape(128, 1), (128, 16))
    att2p = jnp.concatenate([att2[0], jnp.zeros((1,), jnp.float32)])
    attb2 = jnp.broadcast_to(att2p.reshape(48, 1), (48, 16))

    zeros72 = jnp.zeros((STRIPE, 72), jnp.float32)
    zeros48 = jnp.zeros((STRIPE, 48), jnp.float32)

    xl_tbl, rm_tbl = _stage_a(xp, W1l, W1r, att1)
    acc1 = _edge_l1(xl_tbl, rm_tbl, eidx1, attb1, zeros72)
    xl2_tbl, rm2_tbl = _stage_b(acc1, b1, W2l, W2r, att2)
    acc2 = _edge_l2(xl2_tbl, rm2_tbl, eidx2, attb2, zeros48)
    out = _stage_c(acc2, b2)
    return out[:N]


# speedup vs baseline: 27.2738x; 1.4617x over previous
import functools

import jax
import jax.numpy as jnp
from jax import lax
from jax.experimental import pallas as pl
from jax.experimental.pallas import tpu as pltpu
from jax.experimental.pallas import tpu_sc as plsc

N = 10000
E = 320000
D_IN = 128
D_H = 16
HEADS = 8
D_OUT = 47

NPAD = 10112
EPAD = 335872
CHUNK = 128
K1 = EPAD // (16 * CHUNK)
K2 = EPAD // (32 * CHUNK)
STRIPE = NPAD // 16


def _leaky(u):
    return jnp.maximum(u, 0.0) + 0.2 * jnp.minimum(u, 0.0)


def _stage_a_body(x_ref, wl_ref, wr_ref, att_ref, xl_ref, rm_ref):
    x = x_ref[...]
    xl = jnp.dot(x, wl_ref[...], preferred_element_type=jnp.float32)
    xr = jnp.dot(x, wr_ref[...], preferred_element_type=jnp.float32)
    t = _leaky(xl + xr)
    att = att_ref[...]
    mts = [
        jnp.sum(t[:, h * 16:(h + 1) * 16] * att[h][None, :], axis=-1,
                keepdims=True)
        for h in range(HEADS)
    ]
    xl_ref[0] = xl[:, :64]
    xl_ref[1] = xl[:, 64:]
    zero4 = jnp.zeros((NPAD, 4), jnp.float32)
    for c in range(2):
        rm_ref[c] = jnp.concatenate(
            [xr[:, c * 64:(c + 1) * 64]] + mts[c * 4:(c + 1) * 4] + [zero4],
            axis=1)


def _stage_a(xp, w1l, w1r, att1):
    xl, rm = pl.pallas_call(
        _stage_a_body,
        compiler_params=pltpu.CompilerParams(
            vmem_limit_bytes=110 * 1024 * 1024),
        out_shape=(
            jax.ShapeDtypeStruct((2, NPAD, 64), jnp.float32),
            jax.ShapeDtypeStruct((2, NPAD, 72), jnp.float32),
        ),
    )(xp, w1l, w1r, att1)
    return xl.reshape(2 * NPAD, 64), rm.reshape(2 * NPAD, 72)


def _stage_b_body(acc_ref, b1_ref, w2l_ref, w2r_ref, att2_ref,
                  xl2_ref, rm2_ref):
    hs = []
    for c in range(2):
        acc = acc_ref[c]
        for h in range(4):
            z = acc[:, 64 + h:65 + h] + 1e-16
            hs.append(acc[:, h * 16:(h + 1) * 16] / z)
    h1 = jnp.concatenate(hs, axis=1) + b1_ref[...][None, :]
    h1 = jnp.where(h1 > 0, h1, jnp.exp(jnp.minimum(h1, 0.0)) - 1.0)
    xl2 = jnp.dot(h1, w2l_ref[...], preferred_element_type=jnp.float32)
    xr2 = jnp.dot(h1, w2r_ref[...], preferred_element_type=jnp.float32)
    mt2 = jnp.sum(_leaky(xl2 + xr2) * att2_ref[...], axis=-1, keepdims=True)
    xl2_ref[...] = jnp.concatenate([xl2, jnp.ones((NPAD, 1), jnp.float32)],
                                   axis=1)
    rm2_ref[...] = jnp.concatenate([xr2, mt2], axis=1)


def _stage_b(acc1, b1, w2l, w2r, att2):
    return pl.pallas_call(
        _stage_b_body,
        compiler_params=pltpu.CompilerParams(
            vmem_limit_bytes=110 * 1024 * 1024),
        out_shape=(
            jax.ShapeDtypeStruct((NPAD, 48), jnp.float32),
            jax.ShapeDtypeStruct((NPAD, 48), jnp.float32),
        ),
    )(acc1.reshape(2, NPAD, 72), b1, w2l, w2r, att2)


def _stage_c_body(acc_ref, b2_ref, out_ref):
    s = acc_ref[0] + acc_ref[1]
    o = s[:, :47] / (s[:, 47:48] + 1e-16) + b2_ref[...][None, :]
    m = jnp.max(o, axis=1, keepdims=True)
    lse = jnp.log(jnp.sum(jnp.exp(o - m), axis=1, keepdims=True)) + m
    out_ref[...] = o - lse


def _stage_c(acc2, b2):
    return pl.pallas_call(
        _stage_c_body,
        compiler_params=pltpu.CompilerParams(
            vmem_limit_bytes=110 * 1024 * 1024),
        out_shape=jax.ShapeDtypeStruct((NPAD, 47), jnp.float32),
    )(acc2.reshape(2, NPAD, 48), b2)


def _make_edge_kernel(dl, dr, heads, zcol, n_chunks, split_edges_by_core,
                      idx_rows_per_core, att_rows_per_core):
    mesh = plsc.VectorSubcoreMesh(core_axis_name="c", subcore_axis_name="s")
    ngroups = CHUNK // 16
    nblocks = dl // 16 if dl % 16 == 0 else None
    cph = dl // heads
    cover_z = zcol + heads <= dl

    @functools.partial(
        pl.kernel,
        mesh=mesh,
        compiler_params=pltpu.CompilerParams(needs_layout_passes=False,
                                             use_tc_tiling_on_sc=False),
        out_type=jax.ShapeDtypeStruct((2 * NPAD, dr), jnp.float32),
        scratch_types=[
            pltpu.VMEM_SHARED((NPAD, dr), jnp.float32),
            pltpu.VMEM((2, 3, CHUNK), jnp.int32),
            pltpu.VMEM((2, 1, CHUNK), jnp.int32),
            pltpu.VMEM((2, CHUNK, dl), jnp.float32),
            pltpu.VMEM((2, CHUNK, dr), jnp.float32),
            pltpu.VMEM((2, CHUNK, dr), jnp.float32),
            pltpu.VMEM((dl, 16), jnp.float32),
            pltpu.SemaphoreType.DMA,
            pltpu.SemaphoreType.DMA,
        ],
    )
    def edge_kernel(xl_hbm, rm_hbm, eidx_hbm, attb_hbm, zeros_hbm,
                    acc_out, acc_sh, idxb, sidx, rows_l, rows_r, outb,
                    attv, sem_g, sem_s):
        c = lax.axis_index("c")
        s = lax.axis_index("s")
        pltpu.sync_copy(attb_hbm.at[pl.ds(c * att_rows_per_core, dl)], attv)
        pltpu.sync_copy(zeros_hbm, acc_sh.at[pl.ds(s * STRIPE, STRIPE)])
        plsc.subcore_barrier()

        irow0 = c * idx_rows_per_core
        if split_edges_by_core:
            ebase0 = (s * 2 + c) * (n_chunks * CHUNK)
        else:
            ebase0 = s * (n_chunks * CHUNK)
        iot = lax.iota(jnp.int32, 16)
        ones = jnp.full((16,), 1, jnp.int32)

        def load_idx(i, par):
            pltpu.sync_copy(
                eidx_hbm.at[pl.ds(irow0, 3),
                            pl.ds(ebase0 + i * CHUNK, CHUNK)],
                idxb.at[par])

        def start_gathers(par):
            ib = idxb.at[par]
            pltpu.async_copy(xl_hbm.at[ib.at[0]], rows_l.at[par], sem_g)
            pltpu.async_copy(rm_hbm.at[ib.at[1]], rows_r.at[par], sem_g)

        def wait_gathers(par):
            pltpu.make_async_copy(xl_hbm.at[pl.ds(0, CHUNK)],
                                  rows_l.at[par], sem_g).wait()
            pltpu.make_async_copy(rm_hbm.at[pl.ds(0, CHUNK)],
                                  rows_r.at[par], sem_g).wait()

        def wait_scatter(par):
            pltpu.make_async_copy(rm_hbm.at[pl.ds(0, CHUNK)],
                                  outb.at[par], sem_s).wait()

        load_idx(0, 0)
        start_gathers(0)

        def chunk_body(i, _):
            par = lax.rem(i, 2)
            wait_gathers(par)

            @pl.when(i >= 2)
            def _():
                wait_scatter(par)

            for j in range(ngroups):
                sl = pl.ds(j * 16, 16)
                sidx[par, 0, sl] = idxb[par, 2, sl]

            @pl.when(i + 1 < n_chunks)
            def _():
                load_idx(i + 1, 1 - par)
                start_gathers(1 - par)

            rl = rows_l.at[par]
            rr = rows_r.at[par]
            ob = outb.at[par]

            def group_body(g, _):
                rowi = iot + g * 16
                accs = [jnp.zeros((16,), jnp.float32) for _ in range(heads)]
                for b in range(nblocks):
                    h = (b * 16) // cph
                    for d in range(16):
                        cd = jnp.bitwise_and(iot + d, 15) + (b * 16)
                        vl = plsc.load_gather(rl, [rowi, cd])
                        vr = plsc.load_gather(rr, [rowi, cd])
                        u = vl + vr
                        t = jnp.maximum(u, 0.0) + 0.2 * jnp.minimum(u, 0.0)
                        accs[h] = accs[h] + t * attv[b * 16 + d]
                avs = []
                for h in range(heads):
                    mt = plsc.load_gather(
                        rr, [rowi, jnp.full((16,), zcol + h, jnp.int32)])
                    avs.append(jnp.exp(accs[h] - mt))
                for b in range(nblocks):
                    h = (b * 16) // cph
                    for d in range(16):
                        cd = jnp.bitwise_and(iot + d, 15) + (b * 16)
                        vl = plsc.load_gather(rl, [rowi, cd])
                        plsc.store_scatter(ob, [rowi, cd], avs[h] * vl)
                if not cover_z:
                    for h in range(heads):
                        ch = jnp.full((16,), zcol + h, jnp.int32)
                        plsc.store_scatter(ob, [rowi, ch], avs[h])
                return 0

            lax.fori_loop(0, ngroups, group_body, 0)
            pltpu.async_copy(ob, acc_sh.at[sidx.at[par].at[0]], sem_s,
                             add=True)
            return 0

        lax.fori_loop(0, n_chunks, chunk_body, 0)
        wait_scatter(0)
        wait_scatter(1)
        plsc.subcore_barrier()
        pltpu.sync_copy(acc_sh.at[pl.ds(s * STRIPE, STRIPE)],
                        acc_out.at[pl.ds(c * NPAD + s * STRIPE, STRIPE)])

    return edge_kernel


_edge_l1 = _make_edge_kernel(dl=64, dr=72, heads=4, zcol=64, n_chunks=K1,
                             split_edges_by_core=False, idx_rows_per_core=3,
                             att_rows_per_core=64)
_edge_l2 = _make_edge_kernel(dl=48, dr=48, heads=1, zcol=47, n_chunks=K2,
                             split_edges_by_core=True, idx_rows_per_core=0,
                             att_rows_per_core=0)


def kernel(x, edge_index, W1l, W1r, att1, b1, W2l, W2r, att2, b2):
    loop = jnp.arange(N, dtype=edge_index.dtype)
    src = jnp.pad(jnp.concatenate([edge_index[0], loop]), (0, EPAD - E - N),
                  constant_values=N)
    dst = jnp.pad(jnp.concatenate([edge_index[1], loop]), (0, EPAD - E - N),
                  constant_values=N)
    xp = jnp.pad(x, ((0, NPAD - N), (0, 0)))

    eidx1 = jnp.stack([src, dst, dst, src + NPAD, dst + NPAD, dst])
    eidx2 = jnp.stack([src, dst, dst])

    rot = (jnp.arange(16)[:, None] + jnp.arange(16)[None, :]) % 16
    attb1 = jnp.concatenate(
        [att1[h][rot] for h in range(HEADS)], axis=0)
    att2p = jnp.concatenate([att2[0], jnp.zeros((1,), jnp.float32)])
    attb2 = jnp.concatenate(
        [att2p[b * 16 + rot] for b in range(3)], axis=0)

    zeros72 = jnp.zeros((STRIPE, 72), jnp.float32)
    zeros48 = jnp.zeros((STRIPE, 48), jnp.float32)

    xl_tbl, rm_tbl = _stage_a(xp, W1l, W1r, att1)
    acc1 = _edge_l1(xl_tbl, rm_tbl, eidx1, attb1, zeros72)
    xl2_tbl, rm2_tbl = _stage_b(acc1, b1, W2l, W2r, att2)
    acc2 = _edge_l2(xl2_tbl, rm2_tbl, eidx2, attb2, zeros48)
    out = _stage_c(acc2, b2)
    return out[:N]


# speedup vs baseline: 29.0465x; 1.0650x over previous
import functools

import jax
import jax.numpy as jnp
from jax import lax
from jax.experimental import pallas as pl
from jax.experimental.pallas import tpu as pltpu
from jax.experimental.pallas import tpu_sc as plsc

N = 10000
E = 320000
D_IN = 128
D_H = 16
HEADS = 8
D_OUT = 47

NPAD = 10112
EPAD = 335872
CHUNK = 128
K1 = EPAD // (16 * CHUNK)
K2 = EPAD // (32 * CHUNK)
STRIPE = NPAD // 16


def _leaky(u):
    return jnp.maximum(u, 0.0) + 0.2 * jnp.minimum(u, 0.0)


def _stage_a_body(x_ref, wl_ref, wr_ref, att_ref, xl_ref, rm_ref):
    x = x_ref[...]
    xl = jnp.dot(x, wl_ref[...], preferred_element_type=jnp.float32)
    xr = jnp.dot(x, wr_ref[...], preferred_element_type=jnp.float32)
    t = _leaky(xl + xr)
    att = att_ref[...]
    mts = [
        jnp.sum(t[:, h * 16:(h + 1) * 16] * att[h][None, :], axis=-1,
                keepdims=True)
        for h in range(HEADS)
    ]
    xl_ref[0] = xl[:, :64]
    xl_ref[1] = xl[:, 64:]
    zero4 = jnp.zeros((NPAD, 4), jnp.float32)
    for c in range(2):
        rm_ref[c] = jnp.concatenate(
            [xr[:, c * 64:(c + 1) * 64]] + mts[c * 4:(c + 1) * 4] + [zero4],
            axis=1)


def _stage_a(xp, w1l, w1r, att1):
    xl, rm = pl.pallas_call(
        _stage_a_body,
        compiler_params=pltpu.CompilerParams(
            vmem_limit_bytes=110 * 1024 * 1024),
        out_shape=(
            jax.ShapeDtypeStruct((2, NPAD, 64), jnp.float32),
            jax.ShapeDtypeStruct((2, NPAD, 72), jnp.float32),
        ),
    )(xp, w1l, w1r, att1)
    return xl.reshape(2 * NPAD, 64), rm.reshape(2 * NPAD, 72)


def _stage_b_body(acc_ref, b1_ref, w2l_ref, w2r_ref, att2_ref,
                  xl2_ref, rm2_ref):
    hs = []
    for c in range(2):
        acc = acc_ref[c]
        for h in range(4):
            z = acc[:, 64 + h:65 + h] + 1e-16
            hs.append(acc[:, h * 16:(h + 1) * 16] / z)
    h1 = jnp.concatenate(hs, axis=1) + b1_ref[...][None, :]
    h1 = jnp.where(h1 > 0, h1, jnp.exp(jnp.minimum(h1, 0.0)) - 1.0)
    xl2 = jnp.dot(h1, w2l_ref[...], preferred_element_type=jnp.float32)
    xr2 = jnp.dot(h1, w2r_ref[...], preferred_element_type=jnp.float32)
    mt2 = jnp.sum(_leaky(xl2 + xr2) * att2_ref[...], axis=-1, keepdims=True)
    xl2_ref[...] = jnp.concatenate([xl2, jnp.ones((NPAD, 1), jnp.float32)],
                                   axis=1)
    rm2_ref[...] = jnp.concatenate([xr2, mt2], axis=1)


def _stage_b(acc1, b1, w2l, w2r, att2):
    return pl.pallas_call(
        _stage_b_body,
        compiler_params=pltpu.CompilerParams(
            vmem_limit_bytes=110 * 1024 * 1024),
        out_shape=(
            jax.ShapeDtypeStruct((NPAD, 48), jnp.float32),
            jax.ShapeDtypeStruct((NPAD, 48), jnp.float32),
        ),
    )(acc1.reshape(2, NPAD, 72), b1, w2l, w2r, att2)


def _stage_c_body(acc_ref, b2_ref, out_ref):
    s = acc_ref[0] + acc_ref[1]
    o = s[:, :47] / (s[:, 47:48] + 1e-16) + b2_ref[...][None, :]
    m = jnp.max(o, axis=1, keepdims=True)
    lse = jnp.log(jnp.sum(jnp.exp(o - m), axis=1, keepdims=True)) + m
    out_ref[...] = o - lse


def _stage_c(acc2, b2):
    return pl.pallas_call(
        _stage_c_body,
        compiler_params=pltpu.CompilerParams(
            vmem_limit_bytes=110 * 1024 * 1024),
        out_shape=jax.ShapeDtypeStruct((NPAD, 47), jnp.float32),
    )(acc2.reshape(2, NPAD, 48), b2)


def _make_edge_kernel(dl, dr, heads, zcol, n_chunks, split_edges_by_core,
                      idx_rows_per_core, att_rows_per_core):
    mesh = plsc.VectorSubcoreMesh(core_axis_name="c", subcore_axis_name="s")
    ngroups = CHUNK // 16
    nblocks = dl // 16 if dl % 16 == 0 else None
    cph = dl // heads
    cover_z = zcol + heads <= dl

    @functools.partial(
        pl.kernel,
        mesh=mesh,
        compiler_params=pltpu.CompilerParams(needs_layout_passes=False,
                                             use_tc_tiling_on_sc=False),
        out_type=jax.ShapeDtypeStruct((2 * NPAD, dr), jnp.float32),
        scratch_types=[
            pltpu.VMEM_SHARED((NPAD, dr), jnp.float32),
            pltpu.VMEM((2, 3, CHUNK), jnp.int32),
            pltpu.VMEM((2, 1, CHUNK), jnp.int32),
            pltpu.VMEM((2, CHUNK, dl), jnp.float32),
            pltpu.VMEM((2, CHUNK, dr), jnp.float32),
            pltpu.VMEM((2, CHUNK, dr), jnp.float32),
            pltpu.VMEM((dl, 16), jnp.float32),
            pltpu.SemaphoreType.DMA,
            pltpu.SemaphoreType.DMA,
            pltpu.SemaphoreType.DMA,
        ],
    )
    def edge_kernel(xl_hbm, rm_hbm, eidx_hbm, attb_hbm, zeros_hbm,
                    acc_out, acc_sh, idxb, sidx, rows_l, rows_r, outb,
                    attv, sem_g, sem_s, sem_i):
        c = lax.axis_index("c")
        s = lax.axis_index("s")
        pltpu.sync_copy(attb_hbm.at[pl.ds(c * att_rows_per_core, dl)], attv)
        pltpu.sync_copy(zeros_hbm, acc_sh.at[pl.ds(s * STRIPE, STRIPE)])
        plsc.subcore_barrier()

        irow0 = c * idx_rows_per_core
        if split_edges_by_core:
            ebase0 = (s * 2 + c) * (n_chunks * CHUNK)
        else:
            ebase0 = s * (n_chunks * CHUNK)
        iot = lax.iota(jnp.int32, 16)
        ones = jnp.full((16,), 1, jnp.int32)

        def load_idx(i, par):
            pltpu.sync_copy(
                eidx_hbm.at[pl.ds(irow0, 3),
                            pl.ds(ebase0 + i * CHUNK, CHUNK)],
                idxb.at[par])

        def prefetch_idx(i, par):
            pltpu.async_copy(
                eidx_hbm.at[pl.ds(irow0, 3),
                            pl.ds(ebase0 + i * CHUNK, CHUNK)],
                idxb.at[par], sem_i)

        def wait_idx(par):
            pltpu.make_async_copy(
                eidx_hbm.at[pl.ds(0, 3), pl.ds(0, CHUNK)],
                idxb.at[par], sem_i).wait()

        def start_gathers(par):
            ib = idxb.at[par]
            pltpu.async_copy(xl_hbm.at[ib.at[0]], rows_l.at[par], sem_g)
            pltpu.async_copy(rm_hbm.at[ib.at[1]], rows_r.at[par], sem_g)

        def wait_gathers(par):
            pltpu.make_async_copy(xl_hbm.at[pl.ds(0, CHUNK)],
                                  rows_l.at[par], sem_g).wait()
            pltpu.make_async_copy(rm_hbm.at[pl.ds(0, CHUNK)],
                                  rows_r.at[par], sem_g).wait()

        def wait_scatter(par):
            pltpu.make_async_copy(rm_hbm.at[pl.ds(0, CHUNK)],
                                  outb.at[par], sem_s).wait()

        load_idx(0, 0)
        start_gathers(0)
        prefetch_idx(1, 1)

        def chunk_body(i, _):
            par = lax.rem(i, 2)
            wait_gathers(par)

            @pl.when(i >= 2)
            def _():
                wait_scatter(par)

            for j in range(ngroups):
                sl = pl.ds(j * 16, 16)
                sidx[par, 0, sl] = idxb[par, 2, sl]

            @pl.when(i + 2 < n_chunks)
            def _():
                prefetch_idx(i + 2, par)

            @pl.when(i + 1 < n_chunks)
            def _():
                wait_idx(1 - par)
                start_gathers(1 - par)

            rl = rows_l.at[par]
            rr = rows_r.at[par]
            ob = outb.at[par]

            def group_body(g, _):
                rowi = iot + g * 16
                accs = [jnp.zeros((16,), jnp.float32) for _ in range(heads)]
                for b in range(nblocks):
                    h = (b * 16) // cph
                    for d in range(16):
                        cd = jnp.bitwise_and(iot + d, 15) + (b * 16)
                        vl = plsc.load_gather(rl, [rowi, cd])
                        vr = plsc.load_gather(rr, [rowi, cd])
                        u = vl + vr
                        t = jnp.maximum(u, 0.0) + 0.2 * jnp.minimum(u, 0.0)
                        accs[h] = accs[h] + t * attv[b * 16 + d]
                avs = []
                for h in range(heads):
                    mt = plsc.load_gather(
                        rr, [rowi, jnp.full((16,), zcol + h, jnp.int32)])
                    avs.append(jnp.exp(accs[h] - mt))
                for b in range(nblocks):
                    h = (b * 16) // cph
                    for d in range(16):
                        cd = jnp.bitwise_and(iot + d, 15) + (b * 16)
                        vl = plsc.load_gather(rl, [rowi, cd])
                        plsc.store_scatter(ob, [rowi, cd], avs[h] * vl)
                if not cover_z:
                    for h in range(heads):
                        ch = jnp.full((16,), zcol + h, jnp.int32)
                        plsc.store_scatter(ob, [rowi, ch], avs[h])
                return 0

            lax.fori_loop(0, ngroups, group_body, 0)
            pltpu.async_copy(ob, acc_sh.at[sidx.at[par].at[0]], sem_s,
                             add=True)
            return 0

        lax.fori_loop(0, n_chunks, chunk_body, 0)
        wait_scatter(0)
        wait_scatter(1)
        plsc.subcore_barrier()
        pltpu.sync_copy(acc_sh.at[pl.ds(s * STRIPE, STRIPE)],
                        acc_out.at[pl.ds(c * NPAD + s * STRIPE, STRIPE)])

    return edge_kernel


_edge_l1 = _make_edge_kernel(dl=64, dr=72, heads=4, zcol=64, n_chunks=K1,
                             split_edges_by_core=False, idx_rows_per_core=3,
                             att_rows_per_core=64)
_edge_l2 = _make_edge_kernel(dl=48, dr=48, heads=1, zcol=47, n_chunks=K2,
                             split_edges_by_core=True, idx_rows_per_core=0,
                             att_rows_per_core=0)


def kernel(x, edge_index, W1l, W1r, att1, b1, W2l, W2r, att2, b2):
    loop = jnp.arange(N, dtype=edge_index.dtype)
    src = jnp.pad(jnp.concatenate([edge_index[0], loop]), (0, EPAD - E - N),
                  constant_values=N)
    dst = jnp.pad(jnp.concatenate([edge_index[1], loop]), (0, EPAD - E - N),
                  constant_values=N)
    xp = jnp.pad(x, ((0, NPAD - N), (0, 0)))

    eidx1 = jnp.stack([src, dst, dst, src + NPAD, dst + NPAD, dst])
    eidx2 = jnp.stack([src, dst, dst])

    rot = (jnp.arange(16)[:, None] + jnp.arange(16)[None, :]) % 16
    attb1 = jnp.concatenate(
        [att1[h][rot] for h in range(HEADS)], axis=0)
    att2p = jnp.concatenate([att2[0], jnp.zeros((1,), jnp.float32)])
    attb2 = jnp.concatenate(
        [att2p[b * 16 + rot] for b in range(3)], axis=0)

    zeros72 = jnp.zeros((STRIPE, 72), jnp.float32)
    zeros48 = jnp.zeros((STRIPE, 48), jnp.float32)

    xl_tbl, rm_tbl = _stage_a(xp, W1l, W1r, att1)
    acc1 = _edge_l1(xl_tbl, rm_tbl, eidx1, attb1, zeros72)
    xl2_tbl, rm2_tbl = _stage_b(acc1, b1, W2l, W2r, att2)
    acc2 = _edge_l2(xl2_tbl, rm2_tbl, eidx2, attb2, zeros48)
    out = _stage_c(acc2, b2)
    return out[:N]
